# Initial kernel scaffold; baseline (speedup 1.0000x reference)
import jax
import jax.numpy as jnp
from jax.experimental import pallas as pl


def kernel(x, f_ij, idx_i, idx_j, rcut_ij, W1, b1, Wf, bf, W2, b2):
    raise NotImplementedError("write your pallas kernel here")


# speedup vs baseline: 1.5215x; 1.5215x over previous
import functools

import jax
import jax.numpy as jnp
from jax import lax
from jax.experimental import pallas as pl
from jax.experimental.pallas import tpu as pltpu
from jax.experimental.pallas import tpu_sc as plsc

NC = 2
NS = 16
NW = NC * NS
LANES = 16


def _ssp(v):
    return jnp.maximum(v, 0.0) + jnp.log1p(jnp.exp(-jnp.abs(v))) - 0.6931471805599453


def _h_body(x_ref, w1t_ref, b1_ref, o_ref):
    o_ref[...] = jnp.dot(x_ref[...], w1t_ref[...],
                         preferred_element_type=jnp.float32) + b1_ref[...]


def _compute_h(x, W1, b1):
    n, d = x.shape
    blk = 1000
    grid = n // blk
    return pl.pallas_call(
        _h_body,
        grid=(grid,),
        in_specs=[
            pl.BlockSpec((blk, d), lambda i: (i, 0)),
            pl.BlockSpec((d, W1.shape[0]), lambda i: (0, 0)),
            pl.BlockSpec((1, W1.shape[0]), lambda i: (0, 0)),
        ],
        out_specs=pl.BlockSpec((blk, W1.shape[0]), lambda i: (i, 0)),
        out_shape=jax.ShapeDtypeStruct((n, W1.shape[0]), jnp.float32),
    )(x, W1.T, b1.reshape(1, -1))


def _wij_body(f_ref, wft_ref, bf_ref, rc_ref, o_ref):
    u = jnp.dot(f_ref[...], wft_ref[...],
                preferred_element_type=jnp.float32) + bf_ref[...]
    o_ref[...] = _ssp(u) * rc_ref[...]


def _compute_wij(f_ij, Wf, bf, rcut):
    p, r = f_ij.shape
    f = Wf.shape[0]
    blk = 4000
    grid = p // blk
    return pl.pallas_call(
        _wij_body,
        grid=(grid,),
        in_specs=[
            pl.BlockSpec((blk, r), lambda i: (i, 0)),
            pl.BlockSpec((r, f), lambda i: (0, 0)),
            pl.BlockSpec((1, f), lambda i: (0, 0)),
            pl.BlockSpec((blk, 1), lambda i: (i, 0)),
        ],
        out_specs=pl.BlockSpec((blk, f), lambda i: (i, 0)),
        out_shape=jax.ShapeDtypeStruct((p, f), jnp.float32),
    )(f_ij, Wf.T, bf.reshape(1, -1), rcut.reshape(-1, 1))


def _out_body(p0_ref, p1_ref, w2t_ref, b2_ref, o_ref):
    acc = p0_ref[...] + p1_ref[...]
    o_ref[...] = _ssp(jnp.dot(acc, w2t_ref[...],
                              preferred_element_type=jnp.float32) + b2_ref[...])


def _compute_out(p0, p1, W2, b2):
    n, f = p0.shape
    d = W2.shape[0]
    blk = 1000
    grid = n // blk
    return pl.pallas_call(
        _out_body,
        grid=(grid,),
        in_specs=[
            pl.BlockSpec((blk, f), lambda i: (i, 0)),
            pl.BlockSpec((blk, f), lambda i: (i, 0)),
            pl.BlockSpec((f, d), lambda i: (0, 0)),
            pl.BlockSpec((1, d), lambda i: (0, 0)),
        ],
        out_specs=pl.BlockSpec((blk, d), lambda i: (i, 0)),
        out_shape=jax.ShapeDtypeStruct((n, d), jnp.float32),
    )(p0, p1, W2.T, b2.reshape(1, -1))


def _make_sc_scatter(n, d, p, chunk):
    nz_tiles = 10
    n_per_tile = n // nz_tiles
    p_per_tile = p // NW
    n_chunks = p_per_tile // chunk
    mesh = plsc.VectorSubcoreMesh(core_axis_name="c", subcore_axis_name="s")

    @functools.partial(
        pl.kernel,
        out_type=jax.ShapeDtypeStruct((NC, n, d), jnp.float32),
        mesh=mesh,
        scratch_types=[
            pltpu.VMEM((chunk,), jnp.int32),
            pltpu.VMEM((chunk,), jnp.int32),
            pltpu.VMEM((chunk, d), jnp.float32),
            pltpu.VMEM((chunk, d), jnp.float32),
            pltpu.VMEM_SHARED((n, d), jnp.float32),
            pltpu.SemaphoreType.DMA,
        ],
    )
    def sc_kernel(h_hbm, wij_hbm, idxj_hbm, idxi_hbm, zero_hbm, out_hbm,
                  idxj_v, idxi_v, rows_v, wij_v, acc_sh, sem):
        c = lax.axis_index("c")
        s = lax.axis_index("s")
        wid = c * NS + s

        row0 = s * n_per_tile

        @pl.when(s < nz_tiles)
        def _():
            pltpu.sync_copy(zero_hbm.at[pl.ds(row0, n_per_tile)],
                            acc_sh.at[pl.ds(row0, n_per_tile)])

        plsc.subcore_barrier()

        base0 = wid * p_per_tile

        def chunk_body(j, carry):
            base = base0 + j * chunk
            pltpu.sync_copy(idxj_hbm.at[pl.ds(base, chunk)], idxj_v)
            pltpu.sync_copy(idxi_hbm.at[pl.ds(base, chunk)], idxi_v)
            pltpu.async_copy(h_hbm.at[idxj_v], rows_v, sem).wait()
            pltpu.sync_copy(wij_hbm.at[pl.ds(base, chunk)], wij_v)

            def mul_body(rr, carry2):
                for k in range(d // LANES):
                    sl = pl.ds(k * LANES, LANES)
                    rows_v[rr, sl] = rows_v[rr, sl] * wij_v[rr, sl]
                return carry2

            lax.fori_loop(0, chunk, mul_body, 0, unroll=2)
            pltpu.sync_copy(rows_v, acc_sh.at[idxi_v], add=True)
            return carry

        lax.fori_loop(0, n_chunks, chunk_body, 0)
        plsc.subcore_barrier()

        @pl.when(s < nz_tiles)
        def _():
            pltpu.sync_copy(acc_sh.at[pl.ds(row0, n_per_tile)],
                            out_hbm.at[c, pl.ds(row0, n_per_tile)])

    return sc_kernel


def kernel(x, f_ij, idx_i, idx_j, rcut_ij, W1, b1, Wf, bf, W2, b2):
    n, d = x.shape
    p = f_ij.shape[0]
    h = _compute_h(x, W1, b1)
    wij = _compute_wij(f_ij, Wf, bf, rcut_ij)
    zeros = jnp.zeros((n, d), jnp.float32)
    sc = _make_sc_scatter(n, d, p, chunk=80)
    parts = sc(h, wij, idx_j.astype(jnp.int32), idx_i.astype(jnp.int32), zeros)
    return _compute_out(parts[0], parts[1], W2, b2)
